# Initial kernel scaffold; baseline (speedup 1.0000x reference)
import jax
import jax.numpy as jnp
from jax.experimental import pallas as pl


def kernel(x, edge_index, values):
    raise NotImplementedError("write your pallas kernel here")


# speedup vs baseline: 4.7900x; 4.7900x over previous
import functools
import numpy as np
import jax
import jax.numpy as jnp
from jax import lax
from jax.experimental import pallas as pl
from jax.experimental.pallas import tpu as pltpu
from jax.experimental.pallas import tpu_sc as plsc

TAU = 0.3
DEGREE = 20
GAMMA = 1.0
LAMBDA_MAX = 500.0
N_USERS = 50000
N_ITEMS = 50000
N_EDGES = 1600000
BATCH = 32

NC = 2
NS = 16
NW = NC * NS
LANES = 16
EB = 128
EP = ((N_EDGES + NW * EB - 1) // (NW * EB)) * (NW * EB)
EPW = EP // NW
NBLK = EPW // EB

NPAD = 50048
ROWS_PER_TILE = NPAD // NS
ZROWS = 136

FLAT = N_USERS * BATCH
FPW = FLAT // NW
CHUNK = 2000
NCHUNK = FPW // CHUNK

_mesh = plsc.VectorSubcoreMesh(core_axis_name="c", subcore_axis_name="s")
_params = pltpu.CompilerParams(needs_layout_passes=False,
                               use_tc_tiling_on_sc=False)


def _chebyshev_coefficients():
    K = DEGREE
    j = np.arange(K + 1)
    theta = np.pi * (j + 0.5) / (K + 1)
    mid = half = LAMBDA_MAX / 2.0
    lam_nodes = mid + half * np.cos(theta)
    v_max = lam_nodes.max() + 1e-12
    s_tilde = lam_nodes / v_max
    exp = GAMMA / 2.0
    s_gamma = np.power(np.clip(s_tilde.astype(np.float32), 1e-12, None), exp)
    tau_gamma = float(TAU) ** exp
    h = s_gamma / (s_gamma + tau_gamma + 1e-10)
    f_nodes = h.astype(np.float64)
    coeffs = np.zeros(K + 1, dtype=np.float64)
    for k in range(K + 1):
        coeffs[k] = 2.0 / (K + 1) * np.sum(f_nodes * np.cos(k * theta))
    coeffs[0] /= 2.0
    return coeffs.astype(np.float32), np.float32(mid), np.float32(half)


_COEFFS, _MID, _HALF = _chebyshev_coefficients()


def _worker_ids():
    cid = lax.axis_index("c")
    sid = lax.axis_index("s")
    return cid, sid, sid * NC + cid


def _zero_accumulator(acc, zbuf, sid):
    def zrow(r, _):
        zbuf[r, pl.ds(0, LANES)] = jnp.zeros((LANES,), jnp.float32)
        zbuf[r, pl.ds(LANES, LANES)] = jnp.zeros((LANES,), jnp.float32)
        return 0
    lax.fori_loop(0, ZROWS, zrow, 0, unroll=4)
    base = sid * ROWS_PER_TILE

    def zdma(i, _):
        pltpu.sync_copy(zbuf, acc.at[pl.ds(base + i * ZROWS, ZROWS)])
        return 0
    lax.fori_loop(0, ROWS_PER_TILE // ZROWS, zdma, 0)


def _scale_rows(rows_v, vals_v):
    def body(e, _):
        val = plsc.load_gather(vals_v, [jnp.full((LANES,), e, jnp.int32)])
        rows_v[e, pl.ds(0, LANES)] = rows_v[e, pl.ds(0, LANES)] * val
        rows_v[e, pl.ds(LANES, LANES)] = rows_v[e, pl.ds(LANES, LANES)] * val
        return 0
    lax.fori_loop(0, EB, body, 0, unroll=4)


def _scale_rows2(rows_v, rows1_v, vals_v):
    def body(e, _):
        val = plsc.load_gather(vals_v, [jnp.full((LANES,), e, jnp.int32)])
        lo = rows_v[e, pl.ds(0, LANES)] + rows1_v[e, pl.ds(0, LANES)]
        hi = rows_v[e, pl.ds(LANES, LANES)] + rows1_v[e, pl.ds(LANES, LANES)]
        rows_v[e, pl.ds(0, LANES)] = lo * val
        rows_v[e, pl.ds(LANES, LANES)] = hi * val
        return 0
    lax.fori_loop(0, EB, body, 0, unroll=4)


def _drain_accumulator(acc, out_hbm, cid, sid):
    base = sid * ROWS_PER_TILE
    pltpu.sync_copy(
        acc.at[pl.ds(base, ROWS_PER_TILE)],
        out_hbm.at[pl.ds(cid * NPAD + base, ROWS_PER_TILE)],
    )


@functools.partial(
    pl.kernel,
    out_type=jax.ShapeDtypeStruct((NC * NPAD, BATCH), jnp.float32),
    mesh=_mesh,
    scratch_types=dict(
        gidx_v=pltpu.VMEM((EB,), jnp.int32),
        sidx_v=pltpu.VMEM((EB,), jnp.int32),
        vals_v=pltpu.VMEM((EB,), jnp.float32),
        rows_v=pltpu.VMEM((EB, BATCH), jnp.float32),
        zbuf=pltpu.VMEM((ZROWS, BATCH), jnp.float32),
        acc=pltpu.VMEM_SHARED((NPAD, BATCH), jnp.float32),
        sem=pltpu.SemaphoreType.DMA,
    ),
    compiler_params=_params,
)
def _spmv_kernel(t_hbm, gidx_hbm, sidx_hbm, vals_hbm, out_hbm,
                 gidx_v, sidx_v, vals_v, rows_v, zbuf, acc, sem):
    cid, sid, wid = _worker_ids()
    _zero_accumulator(acc, zbuf, sid)
    plsc.subcore_barrier()

    def block(b, _):
        base = wid * EPW + b * EB
        pltpu.sync_copy(gidx_hbm.at[pl.ds(base, EB)], gidx_v)
        pltpu.sync_copy(sidx_hbm.at[pl.ds(base, EB)], sidx_v)
        pltpu.sync_copy(vals_hbm.at[pl.ds(base, EB)], vals_v)
        pltpu.async_copy(t_hbm.at[gidx_v], rows_v, sem).wait()
        _scale_rows(rows_v, vals_v)
        pltpu.sync_copy(rows_v, acc.at[sidx_v], add=True)
        return 0

    lax.fori_loop(0, NBLK, block, 0)
    plsc.subcore_barrier()
    _drain_accumulator(acc, out_hbm, cid, sid)


@functools.partial(
    pl.kernel,
    out_type=jax.ShapeDtypeStruct((NC * NPAD, BATCH), jnp.float32),
    mesh=_mesh,
    scratch_types=dict(
        gidx_v=pltpu.VMEM((EB,), jnp.int32),
        sidx_v=pltpu.VMEM((EB,), jnp.int32),
        vals_v=pltpu.VMEM((EB,), jnp.float32),
        rows_v=pltpu.VMEM((EB, BATCH), jnp.float32),
        rows1_v=pltpu.VMEM((EB, BATCH), jnp.float32),
        zbuf=pltpu.VMEM((ZROWS, BATCH), jnp.float32),
        acc=pltpu.VMEM_SHARED((NPAD, BATCH), jnp.float32),
        sem0=pltpu.SemaphoreType.DMA,
        sem1=pltpu.SemaphoreType.DMA,
    ),
    compiler_params=_params,
)
def _spmv2_kernel(u0_hbm, u1_hbm, gidx_hbm, sidx_hbm, vals_hbm, out_hbm,
                  gidx_v, sidx_v, vals_v, rows_v, rows1_v, zbuf, acc,
                  sem0, sem1):
    cid, sid, wid = _worker_ids()
    _zero_accumulator(acc, zbuf, sid)
    plsc.subcore_barrier()

    def block(b, _):
        base = wid * EPW + b * EB
        pltpu.sync_copy(gidx_hbm.at[pl.ds(base, EB)], gidx_v)
        pltpu.sync_copy(sidx_hbm.at[pl.ds(base, EB)], sidx_v)
        pltpu.sync_copy(vals_hbm.at[pl.ds(base, EB)], vals_v)
        cp0 = pltpu.async_copy(u0_hbm.at[gidx_v], rows_v, sem0)
        cp1 = pltpu.async_copy(u1_hbm.at[gidx_v], rows1_v, sem1)
        cp0.wait()
        cp1.wait()
        _scale_rows2(rows_v, rows1_v, vals_v)
        pltpu.sync_copy(rows_v, acc.at[sidx_v], add=True)
        return 0

    lax.fori_loop(0, NBLK, block, 0)
    plsc.subcore_barrier()
    _drain_accumulator(acc, out_hbm, cid, sid)


@functools.partial(
    pl.kernel,
    out_type=(
        jax.ShapeDtypeStruct((FLAT,), jnp.float32),
        jax.ShapeDtypeStruct((FLAT,), jnp.float32),
    ),
    mesh=_mesh,
    scratch_types=dict(
        bg0=pltpu.VMEM((CHUNK,), jnp.float32),
        bg1=pltpu.VMEM((CHUNK,), jnp.float32),
        btc=pltpu.VMEM((CHUNK,), jnp.float32),
        btp=pltpu.VMEM((CHUNK,), jnp.float32),
        by=pltpu.VMEM((CHUNK,), jnp.float32),
        btn=pltpu.VMEM((CHUNK,), jnp.float32),
        byo=pltpu.VMEM((CHUNK,), jnp.float32),
        coef_v=pltpu.VMEM((5 * LANES,), jnp.float32),
    ),
    compiler_params=_params,
)
def _cheby_update_kernel(g0_hbm, g1_hbm, tc_hbm, tp_hbm, y_hbm, coef_hbm,
                         tn_hbm, yo_hbm,
                         bg0, bg1, btc, btp, by, btn, byo, coef_v):
    _, _, wid = _worker_ids()
    pltpu.sync_copy(coef_hbm, coef_v)

    def chunk(ci, _):
        base = wid * FPW + ci * CHUNK
        pltpu.sync_copy(g0_hbm.at[pl.ds(base, CHUNK)], bg0)
        pltpu.sync_copy(g1_hbm.at[pl.ds(base, CHUNK)], bg1)
        pltpu.sync_copy(tc_hbm.at[pl.ds(base, CHUNK)], btc)
        pltpu.sync_copy(tp_hbm.at[pl.ds(base, CHUNK)], btp)
        pltpu.sync_copy(y_hbm.at[pl.ds(base, CHUNK)], by)

        def vec(i, _):
            sl = pl.ds(i * LANES, LANES)
            a = coef_v[pl.ds(0, LANES)]
            am = coef_v[pl.ds(LANES, LANES)]
            b = coef_v[pl.ds(2 * LANES, LANES)]
            cy = coef_v[pl.ds(3 * LANES, LANES)]
            ck = coef_v[pl.ds(4 * LANES, LANES)]
            g = bg0[sl] + bg1[sl]
            tn = a * g + am * btc[sl] + b * btp[sl]
            btn[sl] = tn
            byo[sl] = cy * by[sl] + ck * tn
            return 0

        lax.fori_loop(0, CHUNK // LANES, vec, 0, unroll=4)
        pltpu.sync_copy(btn, tn_hbm.at[pl.ds(base, CHUNK)])
        pltpu.sync_copy(byo, yo_hbm.at[pl.ds(base, CHUNK)])
        return 0

    lax.fori_loop(0, NCHUNK, chunk, 0)


def _step_coefs(k):
    a = (2.0 if k >= 2 else 1.0) / _HALF
    am = -a * _MID
    b = -1.0 if k >= 2 else 0.0
    cy = 1.0 if k >= 2 else float(_COEFFS[0])
    ck = float(_COEFFS[k])
    row = np.stack([np.full(LANES, s, np.float32)
                    for s in (a, am, b, cy, ck)])
    return row.reshape(-1)


_STEP_COEFS = [None] + [_step_coefs(k) for k in range(1, DEGREE + 1)]


def _gram_parts(t, r_idx, c_idx, vals):
    u_parts = _spmv_kernel(t, c_idx, r_idx, vals)
    g_parts = _spmv2_kernel(u_parts[:N_USERS], u_parts[NPAD:NPAD + N_USERS],
                            r_idx, c_idx, vals)
    return g_parts[:N_ITEMS], g_parts[NPAD:NPAD + N_ITEMS]


@jax.jit
def kernel(x, edge_index, values):
    r_idx = edge_index[0].astype(jnp.int32)
    c_idx = edge_index[1].astype(jnp.int32)
    vals = values.astype(jnp.float32)
    pad = EP - N_EDGES
    r_idx = jnp.pad(r_idx, (0, pad))
    c_idx = jnp.pad(c_idx, (0, pad))
    vals = jnp.pad(vals, (0, pad))

    v = x.T.reshape(N_ITEMS, BATCH)
    v_flat = v.reshape(FLAT)

    g0, g1 = _gram_parts(v, r_idx, c_idx, vals)
    t_cur, y = _cheby_update_kernel(
        g0.reshape(FLAT), g1.reshape(FLAT),
        v_flat, v_flat, v_flat, jnp.asarray(_STEP_COEFS[1]))
    t_prev = v_flat

    for k in range(2, DEGREE + 1):
        g0, g1 = _gram_parts(t_cur.reshape(N_ITEMS, BATCH), r_idx, c_idx,
                             vals)
        t_next, y = _cheby_update_kernel(
            g0.reshape(FLAT), g1.reshape(FLAT),
            t_cur, t_prev, y, jnp.asarray(_STEP_COEFS[k]))
        t_prev, t_cur = t_cur, t_next

    return y.reshape(N_ITEMS, BATCH).T


# speedup vs baseline: 5.7015x; 1.1903x over previous
import functools
import numpy as np
import jax
import jax.numpy as jnp
from jax import lax
from jax.experimental import pallas as pl
from jax.experimental.pallas import tpu as pltpu
from jax.experimental.pallas import tpu_sc as plsc

TAU = 0.3
DEGREE = 20
GAMMA = 1.0
LAMBDA_MAX = 500.0
N_USERS = 50000
N_ITEMS = 50000
N_EDGES = 1600000
BATCH = 32

NC = 2
NS = 16
NW = NC * NS
LANES = 16
EB = 128
SB = 5
NCH = 80
NBLK = SB * NCH
EPW = NBLK * EB
EP = NW * EPW
CEDGE = SB * EB

NPAD = 50048
ROWS_PER_TILE = NPAD // NS
ZROWS = 68

FLAT = N_USERS * BATCH
FPW = FLAT // NW
CHUNK = 10000
NCHUNK = FPW // CHUNK

_mesh = plsc.VectorSubcoreMesh(core_axis_name="c", subcore_axis_name="s")
_params = pltpu.CompilerParams(needs_layout_passes=False,
                               use_tc_tiling_on_sc=False)


def _chebyshev_coefficients():
    K = DEGREE
    j = np.arange(K + 1)
    theta = np.pi * (j + 0.5) / (K + 1)
    mid = half = LAMBDA_MAX / 2.0
    lam_nodes = mid + half * np.cos(theta)
    v_max = lam_nodes.max() + 1e-12
    s_tilde = lam_nodes / v_max
    exp = GAMMA / 2.0
    s_gamma = np.power(np.clip(s_tilde.astype(np.float32), 1e-12, None), exp)
    tau_gamma = float(TAU) ** exp
    h = s_gamma / (s_gamma + tau_gamma + 1e-10)
    f_nodes = h.astype(np.float64)
    coeffs = np.zeros(K + 1, dtype=np.float64)
    for k in range(K + 1):
        coeffs[k] = 2.0 / (K + 1) * np.sum(f_nodes * np.cos(k * theta))
    coeffs[0] /= 2.0
    return coeffs.astype(np.float32), np.float32(mid), np.float32(half)


_COEFFS, _MID, _HALF = _chebyshev_coefficients()


def _worker_ids():
    cid = lax.axis_index("c")
    sid = lax.axis_index("s")
    return cid, sid, sid * NC + cid


def _zero_accumulator(acc, zbuf, sid):
    def zrow(r, _):
        zbuf[r, pl.ds(0, LANES)] = jnp.zeros((LANES,), jnp.float32)
        zbuf[r, pl.ds(LANES, LANES)] = jnp.zeros((LANES,), jnp.float32)
        return 0
    lax.fori_loop(0, ZROWS, zrow, 0, unroll=4)
    base = sid * ROWS_PER_TILE

    def zdma(i, _):
        pltpu.sync_copy(zbuf, acc.at[pl.ds(base + i * ZROWS, ZROWS)])
        return 0
    lax.fori_loop(0, ROWS_PER_TILE // ZROWS, zdma, 0)


def _scale_rows(rows_c, vals_f):
    def body(e, _):
        val = plsc.load_gather(vals_f, [jnp.full((LANES,), e, jnp.int32)])
        rows_c[e, pl.ds(0, LANES)] = rows_c[e, pl.ds(0, LANES)] * val
        rows_c[e, pl.ds(LANES, LANES)] = rows_c[e, pl.ds(LANES, LANES)] * val
        return 0
    lax.fori_loop(0, CEDGE, body, 0, unroll=8)


def _drain_accumulator(acc, out_hbm, cid, sid):
    base = sid * ROWS_PER_TILE
    pltpu.sync_copy(
        acc.at[pl.ds(base, ROWS_PER_TILE)],
        out_hbm.at[pl.ds(cid * NPAD + base, ROWS_PER_TILE)],
    )


def _stage_chunk(wid, ch, gidx2_hbm, sidx2_hbm, vals_hbm, gidx_c, sidx_c,
                 vals_f):
    row0 = wid * NBLK + ch * SB
    pltpu.sync_copy(gidx2_hbm.at[pl.ds(row0, SB)], gidx_c)
    pltpu.sync_copy(sidx2_hbm.at[pl.ds(row0, SB)], sidx_c)
    pltpu.sync_copy(vals_hbm.at[pl.ds(row0 * EB, CEDGE)], vals_f)


def _scatter_chunk(rows_c, sidx_c, acc, sem_s):
    cps = [
        pltpu.async_copy(rows_c.at[pl.ds(g * EB, EB)], acc.at[sidx_c.at[g]],
                         sem_s, add=True)
        for g in range(SB)
    ]
    for cp in cps:
        cp.wait()


@functools.partial(
    pl.kernel,
    out_type=jax.ShapeDtypeStruct((NC * NPAD, BATCH), jnp.float32),
    mesh=_mesh,
    scratch_types=dict(
        gidx_c=pltpu.VMEM((SB, EB), jnp.int32),
        sidx_c=pltpu.VMEM((SB, EB), jnp.int32),
        vals_f=pltpu.VMEM((CEDGE,), jnp.float32),
        rows_c=pltpu.VMEM((CEDGE, BATCH), jnp.float32),
        zbuf=pltpu.VMEM((ZROWS, BATCH), jnp.float32),
        acc=pltpu.VMEM_SHARED((NPAD, BATCH), jnp.float32),
        sem_g=pltpu.SemaphoreType.DMA,
        sem_s=pltpu.SemaphoreType.DMA,
    ),
    compiler_params=_params,
)
def _spmv_kernel(t_hbm, gidx2_hbm, sidx2_hbm, vals_hbm, out_hbm,
                 gidx_c, sidx_c, vals_f, rows_c, zbuf, acc, sem_g, sem_s):
    cid, sid, wid = _worker_ids()
    _zero_accumulator(acc, zbuf, sid)
    plsc.subcore_barrier()

    def chunk(ch, _):
        _stage_chunk(wid, ch, gidx2_hbm, sidx2_hbm, vals_hbm,
                     gidx_c, sidx_c, vals_f)
        cps = [
            pltpu.async_copy(t_hbm.at[gidx_c.at[g]],
                             rows_c.at[pl.ds(g * EB, EB)], sem_g)
            for g in range(SB)
        ]
        for cp in cps:
            cp.wait()
        _scale_rows(rows_c, vals_f)
        _scatter_chunk(rows_c, sidx_c, acc, sem_s)
        return 0

    lax.fori_loop(0, NCH, chunk, 0)
    plsc.subcore_barrier()
    _drain_accumulator(acc, out_hbm, cid, sid)


@functools.partial(
    pl.kernel,
    out_type=jax.ShapeDtypeStruct((FLAT,), jnp.float32),
    mesh=_mesh,
    scratch_types=dict(
        b0=pltpu.VMEM((CHUNK,), jnp.float32),
        b1=pltpu.VMEM((CHUNK,), jnp.float32),
    ),
    compiler_params=_params,
)
def _combine_kernel(u0_hbm, u1_hbm, out_hbm, b0, b1):
    _, _, wid = _worker_ids()

    def chunk(ci, _):
        base = wid * FPW + ci * CHUNK
        pltpu.sync_copy(u0_hbm.at[pl.ds(base, CHUNK)], b0)
        pltpu.sync_copy(u1_hbm.at[pl.ds(base, CHUNK)], b1)

        def vec(i, _):
            sl = pl.ds(i * LANES, LANES)
            b0[sl] = b0[sl] + b1[sl]
            return 0

        lax.fori_loop(0, CHUNK // LANES, vec, 0, unroll=8)
        pltpu.sync_copy(b0, out_hbm.at[pl.ds(base, CHUNK)])
        return 0

    lax.fori_loop(0, NCHUNK, chunk, 0)


@functools.partial(
    pl.kernel,
    out_type=(
        jax.ShapeDtypeStruct((FLAT,), jnp.float32),
        jax.ShapeDtypeStruct((FLAT,), jnp.float32),
    ),
    mesh=_mesh,
    scratch_types=dict(
        bg0=pltpu.VMEM((CHUNK,), jnp.float32),
        bg1=pltpu.VMEM((CHUNK,), jnp.float32),
        btc=pltpu.VMEM((CHUNK,), jnp.float32),
        btp=pltpu.VMEM((CHUNK,), jnp.float32),
        by=pltpu.VMEM((CHUNK,), jnp.float32),
        btn=pltpu.VMEM((CHUNK,), jnp.float32),
        byo=pltpu.VMEM((CHUNK,), jnp.float32),
        coef_v=pltpu.VMEM((5 * LANES,), jnp.float32),
    ),
    compiler_params=_params,
)
def _cheby_update_kernel(g0_hbm, g1_hbm, tc_hbm, tp_hbm, y_hbm, coef_hbm,
                         tn_hbm, yo_hbm,
                         bg0, bg1, btc, btp, by, btn, byo, coef_v):
    _, _, wid = _worker_ids()
    pltpu.sync_copy(coef_hbm, coef_v)

    def chunk(ci, _):
        base = wid * FPW + ci * CHUNK
        pltpu.sync_copy(g0_hbm.at[pl.ds(base, CHUNK)], bg0)
        pltpu.sync_copy(g1_hbm.at[pl.ds(base, CHUNK)], bg1)
        pltpu.sync_copy(tc_hbm.at[pl.ds(base, CHUNK)], btc)
        pltpu.sync_copy(tp_hbm.at[pl.ds(base, CHUNK)], btp)
        pltpu.sync_copy(y_hbm.at[pl.ds(base, CHUNK)], by)

        def vec(i, _):
            sl = pl.ds(i * LANES, LANES)
            a = coef_v[pl.ds(0, LANES)]
            am = coef_v[pl.ds(LANES, LANES)]
            b = coef_v[pl.ds(2 * LANES, LANES)]
            cy = coef_v[pl.ds(3 * LANES, LANES)]
            ck = coef_v[pl.ds(4 * LANES, LANES)]
            g = bg0[sl] + bg1[sl]
            tn = a * g + am * btc[sl] + b * btp[sl]
            btn[sl] = tn
            byo[sl] = cy * by[sl] + ck * tn
            return 0

        lax.fori_loop(0, CHUNK // LANES, vec, 0, unroll=8)
        pltpu.sync_copy(btn, tn_hbm.at[pl.ds(base, CHUNK)])
        pltpu.sync_copy(byo, yo_hbm.at[pl.ds(base, CHUNK)])
        return 0

    lax.fori_loop(0, NCHUNK, chunk, 0)


def _step_coefs(k):
    a = (2.0 if k >= 2 else 1.0) / _HALF
    am = -a * _MID
    b = -1.0 if k >= 2 else 0.0
    cy = 1.0 if k >= 2 else float(_COEFFS[0])
    ck = float(_COEFFS[k])
    row = np.stack([np.full(LANES, s, np.float32)
                    for s in (a, am, b, cy, ck)])
    return row.reshape(-1)


_STEP_COEFS = [None] + [_step_coefs(k) for k in range(1, DEGREE + 1)]


def _gram_parts(t, r2, c2, vals):
    u_parts = _spmv_kernel(t, c2, r2, vals)
    u = _combine_kernel(u_parts[:N_USERS].reshape(FLAT),
                        u_parts[NPAD:NPAD + N_USERS].reshape(FLAT))
    g_parts = _spmv_kernel(u.reshape(N_USERS, BATCH), r2, c2, vals)
    return g_parts[:N_ITEMS], g_parts[NPAD:NPAD + N_ITEMS]


@jax.jit
def kernel(x, edge_index, values):
    r_idx = edge_index[0].astype(jnp.int32)
    c_idx = edge_index[1].astype(jnp.int32)
    vals = values.astype(jnp.float32)
    pad = EP - N_EDGES
    r2 = jnp.pad(r_idx, (0, pad)).reshape(EP // EB, EB)
    c2 = jnp.pad(c_idx, (0, pad)).reshape(EP // EB, EB)
    vals = jnp.pad(vals, (0, pad))

    v = x.T.reshape(N_ITEMS, BATCH)
    v_flat = v.reshape(FLAT)

    g0, g1 = _gram_parts(v, r2, c2, vals)
    t_cur, y = _cheby_update_kernel(
        g0.reshape(FLAT), g1.reshape(FLAT),
        v_flat, v_flat, v_flat, jnp.asarray(_STEP_COEFS[1]))
    t_prev = v_flat

    for k in range(2, DEGREE + 1):
        g0, g1 = _gram_parts(t_cur.reshape(N_ITEMS, BATCH), r2, c2, vals)
        t_next, y = _cheby_update_kernel(
            g0.reshape(FLAT), g1.reshape(FLAT),
            t_cur, t_prev, y, jnp.asarray(_STEP_COEFS[k]))
        t_prev, t_cur = t_cur, t_next

    return y.reshape(N_ITEMS, BATCH).T


# speedup vs baseline: 8.0485x; 1.4117x over previous
import functools
import numpy as np
import jax
import jax.numpy as jnp
from jax import lax
from jax.experimental import pallas as pl
from jax.experimental.pallas import tpu as pltpu
from jax.experimental.pallas import tpu_sc as plsc

TAU = 0.3
DEGREE = 20
GAMMA = 1.0
LAMBDA_MAX = 500.0
N_USERS = 50000
N_ITEMS = 50000
N_EDGES = 1600000
BATCH = 32

NC = 2
NS = 16
NW = NC * NS
LANES = 16
EB = 128
SB = 3
NCH = 132
NBLK = SB * NCH
EPW = NBLK * EB
EP = NW * EPW
CEDGE = SB * EB

NPAD = 50048
ROWS_PER_TILE = NPAD // NS
ZROWS = 68

FLAT = N_USERS * BATCH
FPW = FLAT // NW
CHUNK = 10000
NCHUNK = FPW // CHUNK

_mesh = plsc.VectorSubcoreMesh(core_axis_name="c", subcore_axis_name="s")
_params = pltpu.CompilerParams(needs_layout_passes=False,
                               use_tc_tiling_on_sc=False)


def _chebyshev_coefficients():
    K = DEGREE
    j = np.arange(K + 1)
    theta = np.pi * (j + 0.5) / (K + 1)
    mid = half = LAMBDA_MAX / 2.0
    lam_nodes = mid + half * np.cos(theta)
    v_max = lam_nodes.max() + 1e-12
    s_tilde = lam_nodes / v_max
    exp = GAMMA / 2.0
    s_gamma = np.power(np.clip(s_tilde.astype(np.float32), 1e-12, None), exp)
    tau_gamma = float(TAU) ** exp
    h = s_gamma / (s_gamma + tau_gamma + 1e-10)
    f_nodes = h.astype(np.float64)
    coeffs = np.zeros(K + 1, dtype=np.float64)
    for k in range(K + 1):
        coeffs[k] = 2.0 / (K + 1) * np.sum(f_nodes * np.cos(k * theta))
    coeffs[0] /= 2.0
    return coeffs.astype(np.float32), np.float32(mid), np.float32(half)


_COEFFS, _MID, _HALF = _chebyshev_coefficients()


def _worker_ids():
    cid = lax.axis_index("c")
    sid = lax.axis_index("s")
    return cid, sid, sid * NC + cid


def _zero_accumulator(acc, zbuf, sid):
    def zrow(r, _):
        zbuf[r, pl.ds(0, LANES)] = jnp.zeros((LANES,), jnp.float32)
        zbuf[r, pl.ds(LANES, LANES)] = jnp.zeros((LANES,), jnp.float32)
        return 0
    lax.fori_loop(0, ZROWS, zrow, 0, unroll=4)
    base = sid * ROWS_PER_TILE

    def zdma(i, _):
        pltpu.sync_copy(zbuf, acc.at[pl.ds(base + i * ZROWS, ZROWS)])
        return 0
    lax.fori_loop(0, ROWS_PER_TILE // ZROWS, zdma, 0)


def _scale_rows(rows_c, vals_f):
    def body(e, _):
        val = plsc.load_gather(vals_f, [jnp.full((LANES,), e, jnp.int32)])
        rows_c[e, pl.ds(0, LANES)] = rows_c[e, pl.ds(0, LANES)] * val
        rows_c[e, pl.ds(LANES, LANES)] = rows_c[e, pl.ds(LANES, LANES)] * val
        return 0
    lax.fori_loop(0, CEDGE, body, 0, unroll=8)


def _drain_accumulator(acc, out_hbm, cid, sid):
    base = sid * ROWS_PER_TILE
    pltpu.sync_copy(
        acc.at[pl.ds(base, ROWS_PER_TILE)],
        out_hbm.at[pl.ds(cid * NPAD + base, ROWS_PER_TILE)],
    )


def _stage_chunk(wid, ch, pk_hbm, vals_hbm, idx_c, vals_f):
    row0 = wid * NBLK + ch * SB
    pltpu.sync_copy(pk_hbm.at[pl.ds(row0, SB)], idx_c)
    pltpu.sync_copy(vals_hbm.at[pl.ds(row0 * EB, CEDGE)], vals_f)


def _fire_gathers(t_hbm, idx_c, rows_c, sem_g):
    return [
        pltpu.async_copy(t_hbm.at[idx_c.at[g, 0]],
                         rows_c.at[pl.ds(g * EB, EB)], sem_g)
        for g in range(SB)
    ]


def _fire_scatters(rows_c, idx_c, acc, sem_s):
    return [
        pltpu.async_copy(rows_c.at[pl.ds(g * EB, EB)], acc.at[idx_c.at[g, 1]],
                         sem_s, add=True)
        for g in range(SB)
    ]


@functools.partial(
    pl.kernel,
    out_type=jax.ShapeDtypeStruct((NC * NPAD, BATCH), jnp.float32),
    mesh=_mesh,
    scratch_types=dict(
        idx0=pltpu.VMEM((SB, 2, EB), jnp.int32),
        idx1=pltpu.VMEM((SB, 2, EB), jnp.int32),
        vals0=pltpu.VMEM((CEDGE,), jnp.float32),
        vals1=pltpu.VMEM((CEDGE,), jnp.float32),
        rows0=pltpu.VMEM((CEDGE, BATCH), jnp.float32),
        rows1=pltpu.VMEM((CEDGE, BATCH), jnp.float32),
        zbuf=pltpu.VMEM((ZROWS, BATCH), jnp.float32),
        acc=pltpu.VMEM_SHARED((NPAD, BATCH), jnp.float32),
        sem_g0=pltpu.SemaphoreType.DMA,
        sem_g1=pltpu.SemaphoreType.DMA,
        sem_s0=pltpu.SemaphoreType.DMA,
        sem_s1=pltpu.SemaphoreType.DMA,
    ),
    compiler_params=_params,
)
def _spmv_kernel(t_hbm, pk_hbm, vals_hbm, out_hbm,
                 idx0, idx1, vals0, vals1, rows0, rows1, zbuf, acc,
                 sem_g0, sem_g1, sem_s0, sem_s1):
    cid, sid, wid = _worker_ids()
    _zero_accumulator(acc, zbuf, sid)
    plsc.subcore_barrier()

    def pair(i, _):
        c0 = 2 * i
        _stage_chunk(wid, c0, pk_hbm, vals_hbm, idx0, vals0)
        g0 = _fire_gathers(t_hbm, idx0, rows0, sem_g0)
        _stage_chunk(wid, c0 + 1, pk_hbm, vals_hbm, idx1, vals1)
        g1 = _fire_gathers(t_hbm, idx1, rows1, sem_g1)
        for cp in g0:
            cp.wait()
        _scale_rows(rows0, vals0)
        s0 = _fire_scatters(rows0, idx0, acc, sem_s0)
        for cp in g1:
            cp.wait()
        _scale_rows(rows1, vals1)
        s1 = _fire_scatters(rows1, idx1, acc, sem_s1)
        for cp in s0 + s1:
            cp.wait()
        return 0

    lax.fori_loop(0, NCH // 2, pair, 0)
    plsc.subcore_barrier()
    _drain_accumulator(acc, out_hbm, cid, sid)


@functools.partial(
    pl.kernel,
    out_type=jax.ShapeDtypeStruct((FLAT,), jnp.float32),
    mesh=_mesh,
    scratch_types=dict(
        b0=pltpu.VMEM((CHUNK,), jnp.float32),
        b1=pltpu.VMEM((CHUNK,), jnp.float32),
    ),
    compiler_params=_params,
)
def _combine_kernel(u0_hbm, u1_hbm, out_hbm, b0, b1):
    _, _, wid = _worker_ids()

    def chunk(ci, _):
        base = wid * FPW + ci * CHUNK
        pltpu.sync_copy(u0_hbm.at[pl.ds(base, CHUNK)], b0)
        pltpu.sync_copy(u1_hbm.at[pl.ds(base, CHUNK)], b1)

        def vec(i, _):
            sl = pl.ds(i * LANES, LANES)
            b0[sl] = b0[sl] + b1[sl]
            return 0

        lax.fori_loop(0, CHUNK // LANES, vec, 0, unroll=8)
        pltpu.sync_copy(b0, out_hbm.at[pl.ds(base, CHUNK)])
        return 0

    lax.fori_loop(0, NCHUNK, chunk, 0)


@functools.partial(
    pl.kernel,
    out_type=(
        jax.ShapeDtypeStruct((FLAT,), jnp.float32),
        jax.ShapeDtypeStruct((FLAT,), jnp.float32),
    ),
    mesh=_mesh,
    scratch_types=dict(
        bg0=pltpu.VMEM((CHUNK,), jnp.float32),
        bg1=pltpu.VMEM((CHUNK,), jnp.float32),
        btc=pltpu.VMEM((CHUNK,), jnp.float32),
        btp=pltpu.VMEM((CHUNK,), jnp.float32),
        by=pltpu.VMEM((CHUNK,), jnp.float32),
        btn=pltpu.VMEM((CHUNK,), jnp.float32),
        byo=pltpu.VMEM((CHUNK,), jnp.float32),
        coef_v=pltpu.VMEM((5 * LANES,), jnp.float32),
    ),
    compiler_params=_params,
)
def _cheby_update_kernel(g0_hbm, g1_hbm, tc_hbm, tp_hbm, y_hbm, coef_hbm,
                         tn_hbm, yo_hbm,
                         bg0, bg1, btc, btp, by, btn, byo, coef_v):
    _, _, wid = _worker_ids()
    pltpu.sync_copy(coef_hbm, coef_v)

    def chunk(ci, _):
        base = wid * FPW + ci * CHUNK
        pltpu.sync_copy(g0_hbm.at[pl.ds(base, CHUNK)], bg0)
        pltpu.sync_copy(g1_hbm.at[pl.ds(base, CHUNK)], bg1)
        pltpu.sync_copy(tc_hbm.at[pl.ds(base, CHUNK)], btc)
        pltpu.sync_copy(tp_hbm.at[pl.ds(base, CHUNK)], btp)
        pltpu.sync_copy(y_hbm.at[pl.ds(base, CHUNK)], by)

        def vec(i, _):
            sl = pl.ds(i * LANES, LANES)
            a = coef_v[pl.ds(0, LANES)]
            am = coef_v[pl.ds(LANES, LANES)]
            b = coef_v[pl.ds(2 * LANES, LANES)]
            cy = coef_v[pl.ds(3 * LANES, LANES)]
            ck = coef_v[pl.ds(4 * LANES, LANES)]
            g = bg0[sl] + bg1[sl]
            tn = a * g + am * btc[sl] + b * btp[sl]
            btn[sl] = tn
            byo[sl] = cy * by[sl] + ck * tn
            return 0

        lax.fori_loop(0, CHUNK // LANES, vec, 0, unroll=8)
        pltpu.sync_copy(btn, tn_hbm.at[pl.ds(base, CHUNK)])
        pltpu.sync_copy(byo, yo_hbm.at[pl.ds(base, CHUNK)])
        return 0

    lax.fori_loop(0, NCHUNK, chunk, 0)


def _step_coefs(k):
    a = (2.0 if k >= 2 else 1.0) / _HALF
    am = -a * _MID
    b = -1.0 if k >= 2 else 0.0
    cy = 1.0 if k >= 2 else float(_COEFFS[0])
    ck = float(_COEFFS[k])
    row = np.stack([np.full(LANES, s, np.float32)
                    for s in (a, am, b, cy, ck)])
    return row.reshape(-1)


_STEP_COEFS = [None] + [_step_coefs(k) for k in range(1, DEGREE + 1)]


def _gram_parts(t, pk_a, pk_b, vals):
    u_parts = _spmv_kernel(t, pk_a, vals)
    u = _combine_kernel(u_parts[:N_USERS].reshape(FLAT),
                        u_parts[NPAD:NPAD + N_USERS].reshape(FLAT))
    g_parts = _spmv_kernel(u.reshape(N_USERS, BATCH), pk_b, vals)
    return g_parts[:N_ITEMS], g_parts[NPAD:NPAD + N_ITEMS]


@jax.jit
def kernel(x, edge_index, values):
    r_idx = edge_index[0].astype(jnp.int32)
    c_idx = edge_index[1].astype(jnp.int32)
    vals = values.astype(jnp.float32)
    pad = EP - N_EDGES
    r2 = jnp.pad(r_idx, (0, pad)).reshape(EP // EB, 1, EB)
    c2 = jnp.pad(c_idx, (0, pad)).reshape(EP // EB, 1, EB)
    pk_a = jnp.concatenate([c2, r2], axis=1)
    pk_b = jnp.concatenate([r2, c2], axis=1)
    vals = jnp.pad(vals, (0, pad))

    v = x.T.reshape(N_ITEMS, BATCH)
    v_flat = v.reshape(FLAT)

    g0, g1 = _gram_parts(v, pk_a, pk_b, vals)
    t_cur, y = _cheby_update_kernel(
        g0.reshape(FLAT), g1.reshape(FLAT),
        v_flat, v_flat, v_flat, jnp.asarray(_STEP_COEFS[1]))
    t_prev = v_flat

    for k in range(2, DEGREE + 1):
        g0, g1 = _gram_parts(t_cur.reshape(N_ITEMS, BATCH), pk_a, pk_b, vals)
        t_next, y = _cheby_update_kernel(
            g0.reshape(FLAT), g1.reshape(FLAT),
            t_cur, t_prev, y, jnp.asarray(_STEP_COEFS[k]))
        t_prev, t_cur = t_cur, t_next

    return y.reshape(N_ITEMS, BATCH).T


# speedup vs baseline: 8.0489x; 1.0000x over previous
import functools
import numpy as np
import jax
import jax.numpy as jnp
from jax import lax
from jax.experimental import pallas as pl
from jax.experimental.pallas import tpu as pltpu
from jax.experimental.pallas import tpu_sc as plsc

TAU = 0.3
DEGREE = 20
GAMMA = 1.0
LAMBDA_MAX = 500.0
N_USERS = 50000
N_ITEMS = 50000
N_EDGES = 1600000
BATCH = 32

NC = 2
NS = 16
NW = NC * NS
LANES = 16
EB = 128
SB = 3
NCH = 132
NBLK = SB * NCH
EPW = NBLK * EB
EP = NW * EPW
CEDGE = SB * EB

NPAD = 50048
ROWS_PER_TILE = NPAD // NS
ZROWS = 68

FLAT = N_USERS * BATCH
FPW = FLAT // NW
CHUNK = 10000
NCHUNK = FPW // CHUNK

_mesh = plsc.VectorSubcoreMesh(core_axis_name="c", subcore_axis_name="s")
_params = pltpu.CompilerParams(needs_layout_passes=False,
                               use_tc_tiling_on_sc=False)


def _chebyshev_coefficients():
    K = DEGREE
    j = np.arange(K + 1)
    theta = np.pi * (j + 0.5) / (K + 1)
    mid = half = LAMBDA_MAX / 2.0
    lam_nodes = mid + half * np.cos(theta)
    v_max = lam_nodes.max() + 1e-12
    s_tilde = lam_nodes / v_max
    exp = GAMMA / 2.0
    s_gamma = np.power(np.clip(s_tilde.astype(np.float32), 1e-12, None), exp)
    tau_gamma = float(TAU) ** exp
    h = s_gamma / (s_gamma + tau_gamma + 1e-10)
    f_nodes = h.astype(np.float64)
    coeffs = np.zeros(K + 1, dtype=np.float64)
    for k in range(K + 1):
        coeffs[k] = 2.0 / (K + 1) * np.sum(f_nodes * np.cos(k * theta))
    coeffs[0] /= 2.0
    return coeffs.astype(np.float32), np.float32(mid), np.float32(half)


_COEFFS, _MID, _HALF = _chebyshev_coefficients()


def _worker_ids():
    cid = lax.axis_index("c")
    sid = lax.axis_index("s")
    return cid, sid, sid * NC + cid


def _zero_accumulator(acc, zbuf, sid, sem):
    def zrow(r, _):
        zbuf[r, pl.ds(0, LANES)] = jnp.zeros((LANES,), jnp.float32)
        zbuf[r, pl.ds(LANES, LANES)] = jnp.zeros((LANES,), jnp.float32)
        return 0
    lax.fori_loop(0, ZROWS, zrow, 0, unroll=4)
    base = sid * ROWS_PER_TILE
    def zdma(i, _):
        pltpu.sync_copy(zbuf, acc.at[pl.ds(base + i * ZROWS, ZROWS)])
        return 0
    lax.fori_loop(0, ROWS_PER_TILE // ZROWS, zdma, 0)


def _scale_rows(rows_c, vals_f):
    def body(e, _):
        val = plsc.load_gather(vals_f, [jnp.full((LANES,), e, jnp.int32)])
        rows_c[e, pl.ds(0, LANES)] = rows_c[e, pl.ds(0, LANES)] * val
        rows_c[e, pl.ds(LANES, LANES)] = rows_c[e, pl.ds(LANES, LANES)] * val
        return 0
    lax.fori_loop(0, CEDGE, body, 0, unroll=8)


def _drain_accumulator(acc, out_hbm, cid, sid):
    base = sid * ROWS_PER_TILE
    pltpu.sync_copy(
        acc.at[pl.ds(base, ROWS_PER_TILE)],
        out_hbm.at[pl.ds(cid * NPAD + base, ROWS_PER_TILE)],
    )


def _stage_chunk(wid, ch, pk_hbm, vals_hbm, idx_c, vals_f):
    row0 = wid * NBLK + ch * SB
    pltpu.sync_copy(pk_hbm.at[pl.ds(row0, SB)], idx_c)
    pltpu.sync_copy(vals_hbm.at[pl.ds(row0 * EB, CEDGE)], vals_f)


def _fire_gathers(t_hbm, idx_c, rows_c, sem_g):
    return [
        pltpu.async_copy(t_hbm.at[idx_c.at[g, 0]],
                         rows_c.at[pl.ds(g * EB, EB)], sem_g)
        for g in range(SB)
    ]


def _fire_scatters(rows_c, idx_c, acc, sem_s):
    return [
        pltpu.async_copy(rows_c.at[pl.ds(g * EB, EB)], acc.at[idx_c.at[g, 1]],
                         sem_s, add=True)
        for g in range(SB)
    ]


@functools.partial(
    pl.kernel,
    out_type=jax.ShapeDtypeStruct((NC * NPAD, BATCH), jnp.float32),
    mesh=_mesh,
    scratch_types=dict(
        idx0=pltpu.VMEM((SB, 2, EB), jnp.int32),
        idx1=pltpu.VMEM((SB, 2, EB), jnp.int32),
        vals0=pltpu.VMEM((CEDGE,), jnp.float32),
        vals1=pltpu.VMEM((CEDGE,), jnp.float32),
        rows0=pltpu.VMEM((CEDGE, BATCH), jnp.float32),
        rows1=pltpu.VMEM((CEDGE, BATCH), jnp.float32),
        zbuf=pltpu.VMEM((ZROWS, BATCH), jnp.float32),
        acc=pltpu.VMEM_SHARED((NPAD, BATCH), jnp.float32),
        sem_g0=pltpu.SemaphoreType.DMA,
        sem_g1=pltpu.SemaphoreType.DMA,
        sem_s0=pltpu.SemaphoreType.DMA,
        sem_s1=pltpu.SemaphoreType.DMA,
    ),
    compiler_params=_params,
)
def _spmv_kernel(t_hbm, pk_hbm, vals_hbm, out_hbm,
                 idx0, idx1, vals0, vals1, rows0, rows1, zbuf, acc,
                 sem_g0, sem_g1, sem_s0, sem_s1):
    cid, sid, wid = _worker_ids()
    _zero_accumulator(acc, zbuf, sid, sem_g0)
    plsc.subcore_barrier()

    def pair(i, _):
        c0 = 2 * i
        _stage_chunk(wid, c0, pk_hbm, vals_hbm, idx0, vals0)
        g0 = _fire_gathers(t_hbm, idx0, rows0, sem_g0)
        _stage_chunk(wid, c0 + 1, pk_hbm, vals_hbm, idx1, vals1)
        g1 = _fire_gathers(t_hbm, idx1, rows1, sem_g1)
        for cp in g0:
            cp.wait()
        _scale_rows(rows0, vals0)
        s0 = _fire_scatters(rows0, idx0, acc, sem_s0)
        for cp in g1:
            cp.wait()
        _scale_rows(rows1, vals1)
        s1 = _fire_scatters(rows1, idx1, acc, sem_s1)
        for cp in s0 + s1:
            cp.wait()
        return 0

    lax.fori_loop(0, NCH // 2, pair, 0)
    plsc.subcore_barrier()
    _drain_accumulator(acc, out_hbm, cid, sid)


@functools.partial(
    pl.kernel,
    out_type=jax.ShapeDtypeStruct((FLAT,), jnp.float32),
    mesh=_mesh,
    scratch_types=dict(
        b0=pltpu.VMEM((CHUNK,), jnp.float32),
        b1=pltpu.VMEM((CHUNK,), jnp.float32),
    ),
    compiler_params=_params,
)
def _combine_kernel(u0_hbm, u1_hbm, out_hbm, b0, b1):
    _, _, wid = _worker_ids()

    def chunk(ci, _):
        base = wid * FPW + ci * CHUNK
        pltpu.sync_copy(u0_hbm.at[pl.ds(base, CHUNK)], b0)
        pltpu.sync_copy(u1_hbm.at[pl.ds(base, CHUNK)], b1)

        def vec(i, _):
            sl = pl.ds(i * LANES, LANES)
            b0[sl] = b0[sl] + b1[sl]
            return 0

        lax.fori_loop(0, CHUNK // LANES, vec, 0, unroll=8)
        pltpu.sync_copy(b0, out_hbm.at[pl.ds(base, CHUNK)])
        return 0

    lax.fori_loop(0, NCHUNK, chunk, 0)


@functools.partial(
    pl.kernel,
    out_type=(
        jax.ShapeDtypeStruct((FLAT,), jnp.float32),
        jax.ShapeDtypeStruct((FLAT,), jnp.float32),
    ),
    mesh=_mesh,
    scratch_types=dict(
        bg0=pltpu.VMEM((CHUNK,), jnp.float32),
        bg1=pltpu.VMEM((CHUNK,), jnp.float32),
        btc=pltpu.VMEM((CHUNK,), jnp.float32),
        btp=pltpu.VMEM((CHUNK,), jnp.float32),
        by=pltpu.VMEM((CHUNK,), jnp.float32),
        btn=pltpu.VMEM((CHUNK,), jnp.float32),
        byo=pltpu.VMEM((CHUNK,), jnp.float32),
        coef_v=pltpu.VMEM((5 * LANES,), jnp.float32),
    ),
    compiler_params=_params,
)
def _cheby_update_kernel(g0_hbm, g1_hbm, tc_hbm, tp_hbm, y_hbm, coef_hbm,
                         tn_hbm, yo_hbm,
                         bg0, bg1, btc, btp, by, btn, byo, coef_v):
    _, _, wid = _worker_ids()
    pltpu.sync_copy(coef_hbm, coef_v)

    def chunk(ci, _):
        base = wid * FPW + ci * CHUNK
        pltpu.sync_copy(g0_hbm.at[pl.ds(base, CHUNK)], bg0)
        pltpu.sync_copy(g1_hbm.at[pl.ds(base, CHUNK)], bg1)
        pltpu.sync_copy(tc_hbm.at[pl.ds(base, CHUNK)], btc)
        pltpu.sync_copy(tp_hbm.at[pl.ds(base, CHUNK)], btp)
        pltpu.sync_copy(y_hbm.at[pl.ds(base, CHUNK)], by)

        def vec(i, _):
            sl = pl.ds(i * LANES, LANES)
            a = coef_v[pl.ds(0, LANES)]
            am = coef_v[pl.ds(LANES, LANES)]
            b = coef_v[pl.ds(2 * LANES, LANES)]
            cy = coef_v[pl.ds(3 * LANES, LANES)]
            ck = coef_v[pl.ds(4 * LANES, LANES)]
            g = bg0[sl] + bg1[sl]
            tn = a * g + am * btc[sl] + b * btp[sl]
            btn[sl] = tn
            byo[sl] = cy * by[sl] + ck * tn
            return 0

        lax.fori_loop(0, CHUNK // LANES, vec, 0, unroll=8)
        pltpu.sync_copy(btn, tn_hbm.at[pl.ds(base, CHUNK)])
        pltpu.sync_copy(byo, yo_hbm.at[pl.ds(base, CHUNK)])
        return 0

    lax.fori_loop(0, NCHUNK, chunk, 0)


def _step_coefs(k):
    a = (2.0 if k >= 2 else 1.0) / _HALF
    am = -a * _MID
    b = -1.0 if k >= 2 else 0.0
    cy = 1.0 if k >= 2 else float(_COEFFS[0])
    ck = float(_COEFFS[k])
    row = np.stack([np.full(LANES, s, np.float32)
                    for s in (a, am, b, cy, ck)])
    return row.reshape(-1)


_STEP_COEFS = [None] + [_step_coefs(k) for k in range(1, DEGREE + 1)]


def _gram_parts(t, pk_a, pk_b, vals):
    u_parts = _spmv_kernel(t, pk_a, vals)
    u = _combine_kernel(u_parts[:N_USERS].reshape(FLAT),
                        u_parts[NPAD:NPAD + N_USERS].reshape(FLAT))
    g_parts = _spmv_kernel(u.reshape(N_USERS, BATCH), pk_b, vals)
    return g_parts[:N_ITEMS], g_parts[NPAD:NPAD + N_ITEMS]


@jax.jit
def kernel(x, edge_index, values):
    r_idx = edge_index[0].astype(jnp.int32)
    c_idx = edge_index[1].astype(jnp.int32)
    vals = values.astype(jnp.float32)
    pad = EP - N_EDGES
    r2 = jnp.pad(r_idx, (0, pad)).reshape(EP // EB, 1, EB)
    c2 = jnp.pad(c_idx, (0, pad)).reshape(EP // EB, 1, EB)
    pk_a = jnp.concatenate([c2, r2], axis=1)
    pk_b = jnp.concatenate([r2, c2], axis=1)
    vals = jnp.pad(vals, (0, pad))

    v = x.T.reshape(N_ITEMS, BATCH)
    v_flat = v.reshape(FLAT)

    g0, g1 = _gram_parts(v, pk_a, pk_b, vals)
    t_cur, y = _cheby_update_kernel(
        g0.reshape(FLAT), g1.reshape(FLAT),
        v_flat, v_flat, v_flat, jnp.asarray(_STEP_COEFS[1]))
    t_prev = v_flat

    for k in range(2, DEGREE + 1):
        g0, g1 = _gram_parts(t_cur.reshape(N_ITEMS, BATCH), pk_a, pk_b, vals)
        t_next, y = _cheby_update_kernel(
            g0.reshape(FLAT), g1.reshape(FLAT),
            t_cur, t_prev, y, jnp.asarray(_STEP_COEFS[k]))
        t_prev, t_cur = t_cur, t_next

    return y.reshape(N_ITEMS, BATCH).T
